# Initial kernel scaffold; baseline (speedup 1.0000x reference)
import functools

import jax
import jax.numpy as jnp
from jax import lax
from jax.experimental import pallas as pl
from jax.experimental.pallas import tpu as pltpu
from jax.experimental.pallas import tpu_sc as plsc

NC = 2
NS = 16
LN = 16
EB = 128
CH = 125


def _sc_chebyshev(M, F, NB, xin, rows, cols, vals):
    rpt = M // NS
    nch = rpt // CH

    def body(xin_hbm, rows_hbm, cols_hbm, vals_hbm,
             x1_hbm, x2_hbm, x3_hbm,
             row_v, col_v, val_v, gbuf, stage, acc_sh, sem):
        cid = lax.axis_index("c")
        tid = lax.axis_index("s")

        pltpu.sync_copy(rows_hbm.at[tid], row_v)
        pltpu.sync_copy(cols_hbm.at[tid], col_v)
        pltpu.sync_copy(vals_hbm.at[tid], val_v)

        coff = jnp.full((LN,), cid * M, dtype=jnp.int32)

        def adj_body(j, _):
            for g in range(EB // LN):
                sl = pl.ds(g * LN, LN)
                col_v[j, sl] = col_v[j, sl] + coff
            return 0
        lax.fori_loop(0, NB, adj_body, 0)

        def zero_stage():
            z = jnp.zeros((LN,), jnp.float32)

            def zb(r, _):
                for g in range(F // LN):
                    stage[r, pl.ds(g * LN, LN)] = z
                return 0
            lax.fori_loop(0, CH, zb, 0)

        def scale_batch(j):
            def eb_body(e, _):
                jv = jnp.full((LN,), j, dtype=jnp.int32)
                ev = jnp.full((LN,), e, dtype=jnp.int32)
                vv = plsc.load_gather(val_v, [jv, ev])
                for g in range(F // LN):
                    sl = pl.ds(g * LN, LN)
                    gbuf[e, sl] = gbuf[e, sl] * vv
                return 0
            lax.fori_loop(0, EB, eb_body, 0)

        def step(src_hbm, prev2_hbm, dst_hbm):
            if prev2_hbm is None:
                zero_stage()
                for ch in range(nch):
                    r0 = tid * rpt + ch * CH
                    pltpu.sync_copy(stage, acc_sh.at[pl.ds(r0, CH)])
            else:
                for ch in range(nch):
                    r0 = tid * rpt + ch * CH
                    g0 = cid * M + r0
                    pltpu.sync_copy(prev2_hbm.at[pl.ds(g0, CH)], stage)

                    def neg_body(r, _):
                        for g in range(F // LN):
                            sl = pl.ds(g * LN, LN)
                            stage[r, sl] = -stage[r, sl]
                        return 0
                    lax.fori_loop(0, CH, neg_body, 0)
                    pltpu.sync_copy(stage, acc_sh.at[pl.ds(r0, CH)])
            plsc.subcore_barrier()

            def batch_body(j, _):
                pltpu.async_copy(src_hbm.at[col_v.at[j]], gbuf, sem).wait()
                scale_batch(j)
                pltpu.sync_copy(gbuf, acc_sh.at[row_v.at[j]], add=True)
                return 0
            lax.fori_loop(0, NB, batch_body, 0)
            plsc.subcore_barrier()

            for ch in range(nch):
                r0 = tid * rpt + ch * CH
                g0 = cid * M + r0
                pltpu.sync_copy(acc_sh.at[pl.ds(r0, CH)], stage)
                pltpu.sync_copy(stage, dst_hbm.at[pl.ds(g0, CH)])
            plsc.subcore_barrier()

        step(xin_hbm, None, x1_hbm)

        def dbl_body(j, _):
            for g in range(EB // LN):
                sl = pl.ds(g * LN, LN)
                val_v[j, sl] = val_v[j, sl] * 2.0
            return 0
        lax.fori_loop(0, NB, dbl_body, 0)

        step(x1_hbm, xin_hbm, x2_hbm)
        step(x2_hbm, x1_hbm, x3_hbm)

    out = jax.ShapeDtypeStruct((NC * M, F), jnp.float32)
    fn = pl.kernel(
        body,
        out_type=(out, out, out),
        mesh=plsc.VectorSubcoreMesh(core_axis_name="c", subcore_axis_name="s"),
        scratch_types=[
            pltpu.VMEM((NB, EB), jnp.int32),
            pltpu.VMEM((NB, EB), jnp.int32),
            pltpu.VMEM((NB, EB), jnp.float32),
            pltpu.VMEM((EB, F), jnp.float32),
            pltpu.VMEM((CH, F), jnp.float32),
            pltpu.VMEM_SHARED((M, F), jnp.float32),
            pltpu.SemaphoreType.DMA,
        ],
    )
    return fn(xin, rows, cols, vals)


def _combine_body(x0_ref, x1_ref, x2_ref, x3_ref, w_ref, o_ref):
    acc = jnp.dot(x0_ref[...], w_ref[0], preferred_element_type=jnp.float32)
    acc += jnp.dot(x1_ref[...], w_ref[1], preferred_element_type=jnp.float32)
    acc += jnp.dot(x2_ref[...], w_ref[2], preferred_element_type=jnp.float32)
    acc += jnp.dot(x3_ref[...], w_ref[3], preferred_element_type=jnp.float32)
    o_ref[...] = acc


def _tc_combine(xs, wperm, TM=1000):
    BM, F = xs[0].shape
    Kk, _, Fout = wperm.shape
    xspec = pl.BlockSpec((TM, F), lambda i: (i, 0))
    return pl.pallas_call(
        _combine_body,
        out_shape=jax.ShapeDtypeStruct((BM, Fout), jnp.float32),
        grid=(BM // TM,),
        in_specs=[xspec, xspec, xspec, xspec,
                  pl.BlockSpec((Kk, F, Fout), lambda i: (0, 0, 0))],
        out_specs=pl.BlockSpec((TM, Fout), lambda i: (i, 0)),
    )(*xs, wperm)


def kernel(x, L_indices, L_values, kernel):
    B, M, F = x.shape
    Kk = kernel.shape[0] // F
    Fout = kernel.shape[1]
    NNZ = L_values.shape[0]

    NB = -(-NNZ // (NS * EB))
    pad = NS * NB * EB - NNZ

    row = jnp.concatenate([L_indices[0], jnp.zeros((pad,), jnp.int32)])
    col = jnp.concatenate([L_indices[1], jnp.zeros((pad,), jnp.int32)])
    val = jnp.concatenate([L_values, jnp.zeros((pad,), jnp.float32)])
    rows = row.reshape(NS, NB, EB)
    cols = col.reshape(NS, NB, EB)
    vals = val.reshape(NS, NB, EB)

    xin = x.reshape(B * M, F)
    x1, x2, x3 = _sc_chebyshev(M, F, NB, xin, rows, cols, vals)

    wperm = kernel.reshape(F, Kk, Fout).transpose(1, 0, 2)
    out = _tc_combine([xin, x1, x2, x3], wperm)
    return out.reshape(B, M, Fout)


# speedup vs baseline: 3.9146x; 3.9146x over previous
import functools

import jax
import jax.numpy as jnp
from jax import lax
from jax.experimental import pallas as pl
from jax.experimental.pallas import tpu as pltpu
from jax.experimental.pallas import tpu_sc as plsc

NC = 2
NS = 16
LN = 16
EB = 128
CH = 80


def _sc_chebyshev(M, F, NB, xin, edges):
    nchk = M // CH
    nit = -(-nchk // NS)

    dnums = lax.GatherDimensionNumbers(
        offset_dims=(), collapsed_slice_dims=(0,), start_index_map=(0,))

    def body(xin_hbm, edges_hbm, x1_hbm, x2_hbm, x3_hbm,
             ebuf, cbuf, gbuf, acc_sh, sem):
        cid = lax.axis_index("c")
        tid = lax.axis_index("s")
        coff = jnp.full((LN,), cid * M, dtype=jnp.int32)

        def zero_stage():
            z = jnp.zeros((LN,), jnp.float32)

            def zb(r, _):
                for g in range(F // LN):
                    gbuf[r, pl.ds(g * LN, LN)] = z
                return 0
            lax.fori_loop(0, CH, zb, 0)

        def scale_batch(alpha):
            def grp_body(q, _):
                iv = ebuf[2, pl.ds(q * LN, LN)]
                vals16 = lax.bitcast_convert_type(iv, jnp.float32) * alpha
                for i in range(LN):
                    vv = lax.gather(
                        vals16, jnp.full((LN, 1), i, jnp.int32), dnums,
                        slice_sizes=(1,),
                        mode=lax.GatherScatterMode.PROMISE_IN_BOUNDS)
                    e = q * LN + i
                    for g in range(F // LN):
                        sl = pl.ds(g * LN, LN)
                        gbuf[e, sl] = gbuf[e, sl] * vv
                return 0
            lax.fori_loop(0, EB // LN, grp_body, 0)

        def step(src_hbm, prev2_hbm, dst_hbm, alpha):
            if prev2_hbm is None:
                zero_stage()
                for it in range(nit):
                    k = tid + it * NS

                    @pl.when(k < nchk)
                    def _():
                        pltpu.sync_copy(gbuf.at[pl.ds(0, CH)],
                                        acc_sh.at[pl.ds(k * CH, CH)])
            else:
                for it in range(nit):
                    k = tid + it * NS

                    @pl.when(k < nchk)
                    def _():
                        r0 = k * CH
                        g0 = cid * M + r0
                        pltpu.sync_copy(prev2_hbm.at[pl.ds(g0, CH)],
                                        gbuf.at[pl.ds(0, CH)])

                        def neg_body(r, _):
                            for g in range(F // LN):
                                sl = pl.ds(g * LN, LN)
                                gbuf[r, sl] = -gbuf[r, sl]
                            return 0
                        lax.fori_loop(0, CH, neg_body, 0)
                        pltpu.sync_copy(gbuf.at[pl.ds(0, CH)],
                                        acc_sh.at[pl.ds(r0, CH)])
            plsc.subcore_barrier()

            def batch_body(j, _):
                pltpu.sync_copy(edges_hbm.at[tid, j], ebuf)
                for q in range(EB // LN):
                    sl = pl.ds(q * LN, LN)
                    cbuf[sl] = ebuf[1, sl] + coff
                pltpu.async_copy(src_hbm.at[cbuf], gbuf, sem).wait()
                scale_batch(alpha)
                pltpu.sync_copy(gbuf, acc_sh.at[ebuf.at[0]], add=True)
                return 0
            lax.fori_loop(0, NB, batch_body, 0)
            plsc.subcore_barrier()

            for it in range(nit):
                k = tid + it * NS

                @pl.when(k < nchk)
                def _():
                    r0 = k * CH
                    g0 = cid * M + r0
                    pltpu.sync_copy(acc_sh.at[pl.ds(r0, CH)],
                                    gbuf.at[pl.ds(0, CH)])
                    pltpu.sync_copy(gbuf.at[pl.ds(0, CH)],
                                    dst_hbm.at[pl.ds(g0, CH)])
            plsc.subcore_barrier()

        step(xin_hbm, None, x1_hbm, 1.0)
        step(x1_hbm, xin_hbm, x2_hbm, 2.0)
        step(x2_hbm, x1_hbm, x3_hbm, 2.0)

    out = jax.ShapeDtypeStruct((NC * M, F), jnp.float32)
    fn = pl.kernel(
        body,
        out_type=(out, out, out),
        mesh=plsc.VectorSubcoreMesh(core_axis_name="c", subcore_axis_name="s"),
        scratch_types=[
            pltpu.VMEM((3, EB), jnp.int32),
            pltpu.VMEM((EB,), jnp.int32),
            pltpu.VMEM((EB, F), jnp.float32),
            pltpu.VMEM_SHARED((M, F), jnp.float32),
            pltpu.SemaphoreType.DMA,
        ],
    )
    return fn(xin, edges)


def _combine_body(x0_ref, x1_ref, x2_ref, x3_ref, w_ref, o_ref):
    acc = jnp.dot(x0_ref[...], w_ref[0], preferred_element_type=jnp.float32)
    acc += jnp.dot(x1_ref[...], w_ref[1], preferred_element_type=jnp.float32)
    acc += jnp.dot(x2_ref[...], w_ref[2], preferred_element_type=jnp.float32)
    acc += jnp.dot(x3_ref[...], w_ref[3], preferred_element_type=jnp.float32)
    o_ref[...] = acc


def _tc_combine(xs, wperm, TM=1000):
    BM, F = xs[0].shape
    Kk, _, Fout = wperm.shape
    xspec = pl.BlockSpec((TM, F), lambda i: (i, 0))
    return pl.pallas_call(
        _combine_body,
        out_shape=jax.ShapeDtypeStruct((BM, Fout), jnp.float32),
        grid=(BM // TM,),
        in_specs=[xspec, xspec, xspec, xspec,
                  pl.BlockSpec((Kk, F, Fout), lambda i: (0, 0, 0))],
        out_specs=pl.BlockSpec((TM, Fout), lambda i: (i, 0)),
    )(*xs, wperm)


def kernel(x, L_indices, L_values, kernel):
    B, M, F = x.shape
    Kk = kernel.shape[0] // F
    Fout = kernel.shape[1]
    NNZ = L_values.shape[0]

    NB = -(-NNZ // (NS * EB))
    pad = NS * NB * EB - NNZ

    row = jnp.concatenate([L_indices[0], jnp.zeros((pad,), jnp.int32)])
    col = jnp.concatenate([L_indices[1], jnp.zeros((pad,), jnp.int32)])
    val = jnp.concatenate([L_values, jnp.zeros((pad,), jnp.float32)])
    vbits = lax.bitcast_convert_type(val, jnp.int32)
    edges = jnp.stack([a.reshape(NS, NB, EB) for a in (row, col, vbits)],
                      axis=2)

    xin = x.reshape(B * M, F)
    x1, x2, x3 = _sc_chebyshev(M, F, NB, xin, edges)

    wperm = kernel.reshape(F, Kk, Fout).transpose(1, 0, 2)
    out = _tc_combine([xin, x1, x2, x3], wperm)
    return out.reshape(B, M, Fout)
